# Initial kernel scaffold; baseline (speedup 1.0000x reference)
import jax
import jax.numpy as jnp
from jax.experimental import pallas as pl


def kernel(input, weight):
    raise NotImplementedError("write your pallas kernel here")


# speedup vs baseline: 2.9724x; 2.9724x over previous
import functools

import jax
import jax.numpy as jnp
from jax import lax
from jax.experimental import pallas as pl
from jax.experimental.pallas import tpu as pltpu
from jax.experimental.pallas import tpu_sc as plsc

VOCAB = 100000
EMBED = 128
BATCH = 4096
HIST = 50

_NC = 2
_NS = 16
_NW = _NC * _NS

_B_TOT = BATCH * HIST
_CHUNK = 128
_N_CHUNKS = _B_TOT // _CHUNK
_C_PER_W = _N_CHUNKS // _NW
_B_PER_W = _C_PER_W * _CHUNK

_mesh = plsc.VectorSubcoreMesh(core_axis_name="c", subcore_axis_name="s")


@functools.partial(
    pl.kernel,
    out_type=jax.ShapeDtypeStruct((_B_TOT, EMBED), jnp.float32),
    mesh=_mesh,
    scratch_types=[
        pltpu.VMEM((_C_PER_W, _CHUNK), jnp.int32),
        pltpu.VMEM((_CHUNK, EMBED), jnp.float32),
        pltpu.SemaphoreType.DMA,
    ],
)
def _gather_kernel(table_hbm, idx_hbm, out_hbm, idx_v, buf, sem):
    wid = lax.axis_index("s") * _NC + lax.axis_index("c")
    base = wid * _B_PER_W
    pltpu.sync_copy(idx_hbm.at[wid], idx_v)

    def body(j, carry):
        pltpu.async_copy(table_hbm.at[idx_v.at[j]], buf, sem).wait()
        off = pl.multiple_of(base + j * _CHUNK, _CHUNK)
        pltpu.sync_copy(buf, out_hbm.at[pl.ds(off, _CHUNK)])
        return carry

    lax.fori_loop(0, _C_PER_W, body, 0, unroll=False)


def kernel(input, weight):
    idx = input.reshape(_NW, _C_PER_W, _CHUNK).astype(jnp.int32)
    out = _gather_kernel(weight, idx)
    return out.reshape(BATCH, HIST, EMBED)


# speedup vs baseline: 3.3432x; 1.1248x over previous
import functools

import jax
import jax.numpy as jnp
from jax import lax
from jax.experimental import pallas as pl
from jax.experimental.pallas import tpu as pltpu
from jax.experimental.pallas import tpu_sc as plsc

VOCAB = 100000
EMBED = 128
BATCH = 4096
HIST = 50

_NC = 2
_NS = 16
_NW = _NC * _NS

_B_TOT = BATCH * HIST
_CHUNK = 128
_N_CHUNKS = _B_TOT // _CHUNK
_C_PER_W = _N_CHUNKS // _NW
_B_PER_W = _C_PER_W * _CHUNK
_NBUF = 4

_mesh = plsc.VectorSubcoreMesh(core_axis_name="c", subcore_axis_name="s")


@functools.partial(
    pl.kernel,
    out_type=jax.ShapeDtypeStruct((_B_TOT, EMBED), jnp.float32),
    mesh=_mesh,
    scratch_types=[
        pltpu.VMEM((_C_PER_W, _CHUNK), jnp.int32),
        pltpu.VMEM((_NBUF, _CHUNK, EMBED), jnp.float32),
        pltpu.SemaphoreType.DMA,
        pltpu.SemaphoreType.DMA,
        pltpu.SemaphoreType.DMA,
        pltpu.SemaphoreType.DMA,
        pltpu.SemaphoreType.DMA,
        pltpu.SemaphoreType.DMA,
        pltpu.SemaphoreType.DMA,
        pltpu.SemaphoreType.DMA,
    ],
)
def _gather_kernel(table_hbm, idx_hbm, out_hbm, idx_v, buf,
                   g0, g1, g2, g3, o0, o1, o2, o3):
    gsem = (g0, g1, g2, g3)
    osem = (o0, o1, o2, o3)
    wid = lax.axis_index("s") * _NC + lax.axis_index("c")
    base = wid * _B_PER_W
    pltpu.sync_copy(idx_hbm.at[wid], idx_v)

    def fire_gather(b, j):
        pltpu.async_copy(table_hbm.at[idx_v.at[j]], buf.at[b], gsem[b])

    def drain_gather(b, j):
        pltpu.make_async_copy(table_hbm.at[idx_v.at[j]], buf.at[b],
                              gsem[b]).wait()

    def out_copy(b, j):
        off = pl.multiple_of(base + j * _CHUNK, _CHUNK)
        pltpu.async_copy(buf.at[b], out_hbm.at[pl.ds(off, _CHUNK)], osem[b])

    def drain_out(b, j):
        off = pl.multiple_of(base + j * _CHUNK, _CHUNK)
        pltpu.make_async_copy(buf.at[b], out_hbm.at[pl.ds(off, _CHUNK)],
                              osem[b]).wait()

    for b in range(_NBUF):
        fire_gather(b, b)

    def body(i, carry):
        g = i * _NBUF
        for b in range(_NBUF):
            j = g + b
            drain_gather(b, j)
            out_copy(b, j)
            drain_out(b, j)
            fire_gather(b, j + _NBUF)
        return carry

    lax.fori_loop(0, (_C_PER_W - 2 * _NBUF) // _NBUF + 1, body, 0,
                  unroll=False)

    first_tail = ((_C_PER_W - 2 * _NBUF) // _NBUF + 1) * _NBUF
    for j in range(first_tail, _C_PER_W):
        b = j % _NBUF
        drain_gather(b, j)
        out_copy(b, j)
        drain_out(b, j)
        if j + _NBUF < _C_PER_W:
            fire_gather(b, j + _NBUF)


def kernel(input, weight):
    idx = input.reshape(_NW, _C_PER_W, _CHUNK).astype(jnp.int32)
    out = _gather_kernel(weight, idx)
    return out.reshape(BATCH, HIST, EMBED)


# speedup vs baseline: 5.9409x; 1.7770x over previous
import functools

import jax
import jax.numpy as jnp
from jax import lax
from jax.experimental import pallas as pl
from jax.experimental.pallas import tpu as pltpu
from jax.experimental.pallas import tpu_sc as plsc

VOCAB = 100000
EMBED = 128
BATCH = 4096
HIST = 50
HISTP = 56

_NC = 2
_NS = 16
_NW = _NC * _NS

_B_PER_W = BATCH // _NW
_NBUF = 4

_mesh = plsc.VectorSubcoreMesh(core_axis_name="c", subcore_axis_name="s")


@functools.partial(
    pl.kernel,
    out_type=jax.ShapeDtypeStruct((BATCH, HIST, EMBED), jnp.float32),
    mesh=_mesh,
    scratch_types=[
        pltpu.VMEM((_B_PER_W, HISTP), jnp.int32),
        pltpu.VMEM((_NBUF, HIST, EMBED), jnp.float32),
        pltpu.SemaphoreType.DMA,
        pltpu.SemaphoreType.DMA,
        pltpu.SemaphoreType.DMA,
        pltpu.SemaphoreType.DMA,
        pltpu.SemaphoreType.DMA,
        pltpu.SemaphoreType.DMA,
        pltpu.SemaphoreType.DMA,
        pltpu.SemaphoreType.DMA,
    ],
)
def _gather_kernel(table_hbm, idx_hbm, out_hbm, idx_v, buf,
                   g0, g1, g2, g3, o0, o1, o2, o3):
    gsem = (g0, g1, g2, g3)
    osem = (o0, o1, o2, o3)
    wid = lax.axis_index("s") * _NC + lax.axis_index("c")
    base = pl.multiple_of(wid * _B_PER_W, _B_PER_W)
    pltpu.sync_copy(idx_hbm.at[pl.ds(base, _B_PER_W)], idx_v)

    def fire_gather(b, bi):
        pltpu.async_copy(table_hbm.at[idx_v.at[bi, pl.ds(0, HIST)]],
                         buf.at[b], gsem[b])

    def drain_gather(b, bi):
        pltpu.make_async_copy(table_hbm.at[idx_v.at[bi, pl.ds(0, HIST)]],
                              buf.at[b], gsem[b]).wait()

    def fire_out(b, bi):
        pltpu.async_copy(buf.at[b], out_hbm.at[base + bi], osem[b])

    def drain_out(b, bi):
        pltpu.make_async_copy(buf.at[b], out_hbm.at[base + bi],
                              osem[b]).wait()

    for b in range(_NBUF):
        fire_gather(b, b)

    def body(i, carry):
        g = i * _NBUF
        for b in range(_NBUF):
            bi = g + b
            drain_gather(b, bi)
            fire_out(b, bi)
            drain_out(b, bi)
            fire_gather(b, bi + _NBUF)
        return carry

    n_main = (_B_PER_W - 2 * _NBUF) // _NBUF + 1
    lax.fori_loop(0, n_main, body, 0, unroll=False)

    for bi in range(n_main * _NBUF, _B_PER_W):
        b = bi % _NBUF
        drain_gather(b, bi)
        fire_out(b, bi)
        drain_out(b, bi)
        if bi + _NBUF < _B_PER_W:
            fire_gather(b, bi + _NBUF)


def kernel(input, weight):
    idx = jnp.pad(input.astype(jnp.int32), ((0, 0), (0, HISTP - HIST)))
    return _gather_kernel(weight, idx)


# speedup vs baseline: 5.9517x; 1.0018x over previous
import functools

import jax
import jax.numpy as jnp
from jax import lax
from jax.experimental import pallas as pl
from jax.experimental.pallas import tpu as pltpu
from jax.experimental.pallas import tpu_sc as plsc

VOCAB = 100000
EMBED = 128
BATCH = 4096
HIST = 50
HISTP = 56

_NC = 2
_NS = 16
_NW = _NC * _NS

_B_PER_W = BATCH // _NW
_NBUF = 4

_mesh = plsc.VectorSubcoreMesh(core_axis_name="c", subcore_axis_name="s")


@functools.partial(
    pl.kernel,
    out_type=jax.ShapeDtypeStruct((BATCH, HIST, EMBED), jnp.float32),
    mesh=_mesh,
    compiler_params=pltpu.CompilerParams(use_tc_tiling_on_sc=True),
    scratch_types=[
        pltpu.VMEM((_B_PER_W, HISTP), jnp.int32),
        pltpu.VMEM((_NBUF, HIST, EMBED), jnp.float32),
        pltpu.SemaphoreType.DMA,
        pltpu.SemaphoreType.DMA,
        pltpu.SemaphoreType.DMA,
        pltpu.SemaphoreType.DMA,
        pltpu.SemaphoreType.DMA,
        pltpu.SemaphoreType.DMA,
        pltpu.SemaphoreType.DMA,
        pltpu.SemaphoreType.DMA,
    ],
)
def _gather_kernel(table_hbm, idx_hbm, out_hbm, idx_v, buf,
                   g0, g1, g2, g3, o0, o1, o2, o3):
    gsem = (g0, g1, g2, g3)
    osem = (o0, o1, o2, o3)
    wid = lax.axis_index("s") * _NC + lax.axis_index("c")
    base = pl.multiple_of(wid * _B_PER_W, _B_PER_W)
    pltpu.sync_copy(idx_hbm.at[pl.ds(base, _B_PER_W)], idx_v)

    def fire_gather(b, bi):
        pltpu.async_copy(table_hbm.at[idx_v.at[bi, pl.ds(0, HIST)]],
                         buf.at[b], gsem[b])

    def drain_gather(b, bi):
        pltpu.make_async_copy(table_hbm.at[idx_v.at[bi, pl.ds(0, HIST)]],
                              buf.at[b], gsem[b]).wait()

    def fire_out(b, bi):
        pltpu.async_copy(buf.at[b], out_hbm.at[base + bi], osem[b])

    def drain_out(b, bi):
        pltpu.make_async_copy(buf.at[b], out_hbm.at[base + bi],
                              osem[b]).wait()

    for b in range(_NBUF):
        fire_gather(b, b)

    def body(i, carry):
        g = i * _NBUF
        for b in range(_NBUF):
            bi = g + b
            drain_gather(b, bi)
            fire_out(b, bi)
            drain_out(b, bi)
            fire_gather(b, bi + _NBUF)
        return carry

    n_main = (_B_PER_W - 2 * _NBUF) // _NBUF + 1
    lax.fori_loop(0, n_main, body, 0, unroll=False)

    for bi in range(n_main * _NBUF, _B_PER_W):
        b = bi % _NBUF
        drain_gather(b, bi)
        fire_out(b, bi)
        drain_out(b, bi)
        if bi + _NBUF < _B_PER_W:
            fire_gather(b, bi + _NBUF)


def kernel(input, weight):
    idx = jnp.pad(input.astype(jnp.int32), ((0, 0), (0, HISTP - HIST)))
    return _gather_kernel(weight, idx)


# speedup vs baseline: 10.8219x; 1.8183x over previous
import functools

import jax
import jax.numpy as jnp
from jax import lax
from jax.experimental import pallas as pl
from jax.experimental.pallas import tpu as pltpu
from jax.experimental.pallas import tpu_sc as plsc

VOCAB = 100000
EMBED = 128
BATCH = 4096
HIST = 50

_NC = 2
_NS = 16
_NW = _NC * _NS

_B_PER_W = BATCH // _NW
_NBUF = 7
_GAP = 5
_OUT_LAG = _NBUF - _GAP

_mesh = plsc.VectorSubcoreMesh(core_axis_name="c", subcore_axis_name="s")


@functools.partial(
    pl.kernel,
    out_type=jax.ShapeDtypeStruct((HIST, BATCH, EMBED), jnp.float32),
    mesh=_mesh,
    scratch_types=[
        pltpu.VMEM((HIST, _B_PER_W), jnp.int32),
        pltpu.VMEM((_NBUF, _B_PER_W, EMBED), jnp.float32),
        pltpu.SemaphoreType.DMA,
        pltpu.SemaphoreType.DMA,
        pltpu.SemaphoreType.DMA,
        pltpu.SemaphoreType.DMA,
        pltpu.SemaphoreType.DMA,
        pltpu.SemaphoreType.DMA,
        pltpu.SemaphoreType.DMA,
        pltpu.SemaphoreType.DMA,
        pltpu.SemaphoreType.DMA,
        pltpu.SemaphoreType.DMA,
        pltpu.SemaphoreType.DMA,
        pltpu.SemaphoreType.DMA,
        pltpu.SemaphoreType.DMA,
        pltpu.SemaphoreType.DMA,
    ],
)
def _gather_kernel(table_hbm, idx_hbm, out_hbm, idx_v, buf,
                   g0, g1, g2, g3, g4, g5, g6, o0, o1, o2, o3, o4, o5, o6):
    gsem = (g0, g1, g2, g3, g4, g5, g6)
    osem = (o0, o1, o2, o3, o4, o5, o6)
    wid = lax.axis_index("s") * _NC + lax.axis_index("c")
    base = pl.multiple_of(wid * _B_PER_W, _B_PER_W)
    pltpu.sync_copy(idx_hbm.at[:, pl.ds(base, _B_PER_W)], idx_v)

    def fire_gather(b, h):
        pltpu.async_copy(table_hbm.at[idx_v.at[h]], buf.at[b], gsem[b])

    def drain_gather(b, h):
        pltpu.make_async_copy(table_hbm.at[idx_v.at[h]], buf.at[b],
                              gsem[b]).wait()

    def fire_out(b, h):
        pltpu.async_copy(buf.at[b], out_hbm.at[h, pl.ds(base, _B_PER_W)],
                         osem[b])

    def drain_out(b, h):
        pltpu.make_async_copy(buf.at[b], out_hbm.at[h, pl.ds(base, _B_PER_W)],
                              osem[b]).wait()

    for c in range(_GAP):
        fire_gather(c, c)

    for v in range(_OUT_LAG):
        drain_gather(v % _NBUF, v)
        fire_out(v % _NBUF, v)
        fire_gather((v + _GAP) % _NBUF, v + _GAP)

    def group(i, carry):
        v0 = _OUT_LAG + i * _NBUF
        for k in range(_NBUF):
            v = v0 + k
            s1 = (_OUT_LAG + k) % _NBUF
            s2 = k
            drain_gather(s1, v)
            fire_out(s1, v)
            drain_out(s2, v - _OUT_LAG)
            fire_gather(s2, v + _GAP)
        return carry

    n_groups = (HIST - _GAP - _OUT_LAG) // _NBUF
    lax.fori_loop(0, n_groups, group, 0, unroll=False)

    for v in range(_OUT_LAG + n_groups * _NBUF, HIST):
        s1 = v % _NBUF
        s2 = (v + _GAP) % _NBUF
        drain_gather(s1, v)
        fire_out(s1, v)
        drain_out(s2, v - _OUT_LAG)
        if v + _GAP < HIST:
            fire_gather(s2, v + _GAP)
    for c in range(HIST - _OUT_LAG, HIST):
        drain_out(c % _NBUF, c)


def kernel(input, weight):
    idx = jnp.transpose(input.astype(jnp.int32))
    out = _gather_kernel(weight, idx)
    return jnp.transpose(out, (1, 0, 2))


# speedup vs baseline: 10.8365x; 1.0013x over previous
import functools

import jax
import jax.numpy as jnp
from jax import lax
from jax.experimental import pallas as pl
from jax.experimental.pallas import tpu as pltpu
from jax.experimental.pallas import tpu_sc as plsc

VOCAB = 100000
EMBED = 128
BATCH = 4096
HIST = 50

_NC = 2
_NS = 16
_NW = _NC * _NS

_B_PER_W = BATCH // _NW
_NBUF = 7
_GAP = 5
_OUT_LAG = _NBUF - _GAP

_mesh = plsc.VectorSubcoreMesh(core_axis_name="c", subcore_axis_name="s")


@functools.partial(
    pl.kernel,
    out_type=jax.ShapeDtypeStruct((HIST, BATCH, EMBED), jnp.float32),
    mesh=_mesh,
    compiler_params=pltpu.CompilerParams(disable_bounds_checks=True),
    scratch_types=[
        pltpu.VMEM((HIST, _B_PER_W), jnp.int32),
        pltpu.VMEM((_NBUF, _B_PER_W, EMBED), jnp.float32),
        pltpu.SemaphoreType.DMA,
        pltpu.SemaphoreType.DMA,
        pltpu.SemaphoreType.DMA,
        pltpu.SemaphoreType.DMA,
        pltpu.SemaphoreType.DMA,
        pltpu.SemaphoreType.DMA,
        pltpu.SemaphoreType.DMA,
        pltpu.SemaphoreType.DMA,
        pltpu.SemaphoreType.DMA,
        pltpu.SemaphoreType.DMA,
        pltpu.SemaphoreType.DMA,
        pltpu.SemaphoreType.DMA,
        pltpu.SemaphoreType.DMA,
        pltpu.SemaphoreType.DMA,
    ],
)
def _gather_kernel(table_hbm, idx_hbm, out_hbm, idx_v, buf,
                   g0, g1, g2, g3, g4, g5, g6, o0, o1, o2, o3, o4, o5, o6):
    gsem = (g0, g1, g2, g3, g4, g5, g6)
    osem = (o0, o1, o2, o3, o4, o5, o6)
    wid = lax.axis_index("s") * _NC + lax.axis_index("c")
    base = pl.multiple_of(wid * _B_PER_W, _B_PER_W)
    pltpu.sync_copy(idx_hbm.at[:, pl.ds(base, _B_PER_W)], idx_v)

    def fire_gather(b, h):
        pltpu.async_copy(table_hbm.at[idx_v.at[h]], buf.at[b], gsem[b])

    def drain_gather(b, h):
        pltpu.make_async_copy(table_hbm.at[idx_v.at[h]], buf.at[b],
                              gsem[b]).wait()

    def fire_out(b, h):
        pltpu.async_copy(buf.at[b], out_hbm.at[h, pl.ds(base, _B_PER_W)],
                         osem[b])

    def drain_out(b, h):
        pltpu.make_async_copy(buf.at[b], out_hbm.at[h, pl.ds(base, _B_PER_W)],
                              osem[b]).wait()

    for c in range(_GAP):
        fire_gather(c, c)

    for v in range(_OUT_LAG):
        drain_gather(v % _NBUF, v)
        fire_out(v % _NBUF, v)
        fire_gather((v + _GAP) % _NBUF, v + _GAP)

    def group(i, carry):
        v0 = _OUT_LAG + i * _NBUF
        for k in range(_NBUF):
            v = v0 + k
            s1 = (_OUT_LAG + k) % _NBUF
            s2 = k
            drain_gather(s1, v)
            fire_out(s1, v)
            drain_out(s2, v - _OUT_LAG)
            fire_gather(s2, v + _GAP)
        return carry

    n_groups = (HIST - _GAP - _OUT_LAG) // _NBUF
    lax.fori_loop(0, n_groups, group, 0, unroll=False)

    for v in range(_OUT_LAG + n_groups * _NBUF, HIST):
        s1 = v % _NBUF
        s2 = (v + _GAP) % _NBUF
        drain_gather(s1, v)
        fire_out(s1, v)
        drain_out(s2, v - _OUT_LAG)
        if v + _GAP < HIST:
            fire_gather(s2, v + _GAP)
    for c in range(HIST - _OUT_LAG, HIST):
        drain_out(c % _NBUF, c)


def kernel(input, weight):
    idx = jnp.transpose(input.astype(jnp.int32))
    out = _gather_kernel(weight, idx)
    return jnp.transpose(out, (1, 0, 2))


# speedup vs baseline: 10.8450x; 1.0008x over previous
import functools

import jax
import jax.numpy as jnp
from jax import lax
from jax.experimental import pallas as pl
from jax.experimental.pallas import tpu as pltpu
from jax.experimental.pallas import tpu_sc as plsc

VOCAB = 100000
EMBED = 128
BATCH = 4096
HIST = 50

_NC = 2
_NS = 16
_NW = _NC * _NS

_B_PER_W = BATCH // _NW
_NBUF = 7
_GAP = 5
_OUT_LAG = _NBUF - _GAP

_mesh = plsc.VectorSubcoreMesh(core_axis_name="c", subcore_axis_name="s")


@functools.partial(
    pl.kernel,
    out_type=jax.ShapeDtypeStruct((HIST, BATCH, EMBED), jnp.float32),
    mesh=_mesh,
    scratch_types=[
        pltpu.VMEM((HIST, _B_PER_W), jnp.int32),
        pltpu.VMEM((_NBUF, _B_PER_W, EMBED), jnp.float32),
        pltpu.SemaphoreType.DMA,
        pltpu.SemaphoreType.DMA,
        pltpu.SemaphoreType.DMA,
        pltpu.SemaphoreType.DMA,
        pltpu.SemaphoreType.DMA,
        pltpu.SemaphoreType.DMA,
        pltpu.SemaphoreType.DMA,
        pltpu.SemaphoreType.DMA,
        pltpu.SemaphoreType.DMA,
        pltpu.SemaphoreType.DMA,
        pltpu.SemaphoreType.DMA,
        pltpu.SemaphoreType.DMA,
        pltpu.SemaphoreType.DMA,
        pltpu.SemaphoreType.DMA,
    ],
)
def _gather_kernel(table_hbm, idx_hbm, out_hbm, idx_v, buf,
                   g0, g1, g2, g3, g4, g5, g6, o0, o1, o2, o3, o4, o5, o6):
    gsem = (g0, g1, g2, g3, g4, g5, g6)
    osem = (o0, o1, o2, o3, o4, o5, o6)
    wid = lax.axis_index("s") * _NC + lax.axis_index("c")
    base = pl.multiple_of(wid * _B_PER_W, _B_PER_W)
    pltpu.sync_copy(idx_hbm.at[:, pl.ds(base, _B_PER_W)], idx_v)

    def fire_gather(b, h):
        pltpu.async_copy(table_hbm.at[idx_v.at[h]], buf.at[b], gsem[b])

    def drain_gather(b, h):
        pltpu.make_async_copy(table_hbm.at[idx_v.at[h]], buf.at[b],
                              gsem[b]).wait()

    def fire_out(b, h):
        pltpu.async_copy(buf.at[b], out_hbm.at[h, pl.ds(base, _B_PER_W)],
                         osem[b])

    def drain_out(b, h):
        pltpu.make_async_copy(buf.at[b], out_hbm.at[h, pl.ds(base, _B_PER_W)],
                              osem[b]).wait()

    for c in range(_GAP):
        fire_gather(c, c)

    for v in range(_OUT_LAG):
        drain_gather(v % _NBUF, v)
        fire_out(v % _NBUF, v)
        fire_gather((v + _GAP) % _NBUF, v + _GAP)

    def group(i, carry):
        v0 = _OUT_LAG + i * _NBUF
        for k in range(_NBUF):
            v = v0 + k
            s1 = (_OUT_LAG + k) % _NBUF
            s2 = k
            drain_gather(s1, v)
            fire_out(s1, v)
            drain_out(s2, v - _OUT_LAG)
            fire_gather(s2, v + _GAP)
        return carry

    n_groups = (HIST - _GAP - _OUT_LAG) // _NBUF
    lax.fori_loop(0, n_groups, group, 0, unroll=False)

    for v in range(_OUT_LAG + n_groups * _NBUF, HIST):
        s1 = v % _NBUF
        s2 = (v + _GAP) % _NBUF
        drain_gather(s1, v)
        fire_out(s1, v)
        drain_out(s2, v - _OUT_LAG)
        if v + _GAP < HIST:
            fire_gather(s2, v + _GAP)
    for c in range(HIST - _OUT_LAG, HIST):
        drain_out(c % _NBUF, c)


def kernel(input, weight):
    idx = jnp.transpose(input.astype(jnp.int32))
    out = _gather_kernel(weight, idx)
    return jnp.transpose(out, (1, 0, 2))
